# baseline (speedup 1.0000x reference)
import functools

import jax
import jax.numpy as jnp
from jax.experimental import pallas as pl
from jax.experimental.pallas import tpu as pltpu

E = 8
TOP = 2
C = 192
G = 4
B = 8
H = 16
W = 16
CCAP = 192
HW = H * W
BHW = B * HW


def _moe_body(x_ref, gates_ref, wc_ref, bc_ref, wp_ref, bp_ref,
              y0_ref, y1_ref, y2_ref, y3_ref, loss_ref, xs_ref, wr_ref):
    e = pl.program_id(0)
    y_refs = [y0_ref, y1_ref, y2_ref, y3_ref]

    @pl.when(e == 0)
    def _prologue():
        xT = jnp.concatenate([x_ref[b] for b in range(B)], axis=1)
        rio = jax.lax.broadcasted_iota(jnp.int32, (1, BHW), 1)
        pool = (jax.lax.broadcasted_iota(jnp.int32, (BHW, B), 0) // HW ==
                jax.lax.broadcasted_iota(jnp.int32, (BHW, B), 1))
        poolf = pool.astype(jnp.float32)
        x_gap = jnp.dot(xT, poolf,
                        preferred_element_type=jnp.float32) * (1.0 / HW)
        eio = jax.lax.broadcasted_iota(jnp.int32, (E, B), 0)
        loss_acc = jnp.float32(0.0)
        for g in range(G):
            logits = jnp.dot(gates_ref[g], x_gap,
                             preferred_element_type=jnp.float32)
            m = jnp.max(logits, axis=0, keepdims=True)
            ex = jnp.exp(logits - m)
            probs = ex / jnp.sum(ex, axis=0, keepdims=True)
            usage = jnp.sum(probs, axis=1)
            mu = jnp.mean(usage)
            var = jnp.mean((usage - mu) ** 2)
            loss_acc = loss_acc + var / (mu * mu + 1e-10)
            v1 = jnp.max(probs, axis=0, keepdims=True)
            i1 = jnp.min(jnp.where(probs == v1, eio, E + 1), axis=0,
                         keepdims=True)
            p2 = jnp.where(eio == i1, -1.0, probs)
            v2 = jnp.max(p2, axis=0, keepdims=True)
            i2 = jnp.min(jnp.where(p2 == v2, eio, E + 1), axis=0,
                         keepdims=True)
            t = jnp.exp(v2 - v1)
            w1 = 1.0 / (1.0 + t)
            w2 = t / (1.0 + t)
            cw = jnp.where(eio == i1, w1, jnp.float32(0.0)) \
                + jnp.where(eio == i2, w2, jnp.float32(0.0))
            for e_ in range(E):
                wr_ref[e_, g:g + 1, :] = jnp.concatenate(
                    [jnp.broadcast_to(cw[e_:e_ + 1, b:b + 1], (1, HW))
                     for b in range(B)], axis=1)
        loss_ref[...] = jnp.broadcast_to(loss_acc / G, (1, 1))

        xb = xT.astype(jnp.bfloat16)
        hpos = (rio // W) % H
        wpos = rio % W
        for dy in range(3):
            for dx in range(3):
                k = dy * 3 + dx
                sh, sw = dy - 1, dx - 1
                shift = sh * W + sw
                rolled = jnp.roll(xb, -shift, axis=1) if shift != 0 else xb
                mask = jnp.ones((1, BHW), jnp.bool_)
                if sh > 0:
                    mask = mask & (hpos < H - sh)
                elif sh < 0:
                    mask = mask & (hpos >= -sh)
                if sw > 0:
                    mask = mask & (wpos < W - sw)
                elif sw < 0:
                    mask = mask & (wpos >= -sw)
                xs_ref[k * C:(k + 1) * C, :] = rolled * mask.astype(jnp.bfloat16)

    ones_row = jnp.ones((1, CCAP), jnp.bfloat16)
    u = jnp.dot(wc_ref[0], xs_ref[...], preferred_element_type=jnp.float32)
    u = u + bc_ref[0]
    ub = u.astype(jnp.bfloat16)
    sn = jnp.dot(ones_row, ub * ub,
                 preferred_element_type=jnp.float32)
    scale = sn / ((1.0 + sn) * (jnp.sqrt(sn) + 1e-8))
    uq = (scale * u).astype(jnp.bfloat16)
    out = jnp.dot(wp_ref[0].astype(jnp.bfloat16), uq,
                  preferred_element_type=jnp.float32) + bp_ref[0]
    for g in range(G):
        contrib = wr_ref[e, g:g + 1, :] * out
        for b in range(B):
            blk = contrib[:, b * HW:(b + 1) * HW]

            @pl.when(e == 0)
            def _init(b=b, g=g, blk=blk):
                y_refs[g][b] = blk

            @pl.when(e > 0)
            def _acc(b=b, g=g, blk=blk):
                y_refs[g][b] = y_refs[g][b] + blk


@jax.jit
def _moe(x, Wc, bc, Wp, bp, gates):
    x_nat = x.reshape(B, C, HW)
    Wc_r = jnp.transpose(Wc.astype(jnp.bfloat16),
                         (0, 1, 3, 4, 2)).reshape(E, CCAP, 9 * C)
    bc_r = bc.reshape(E, CCAP, 1)
    Wp_r = Wp[..., 0, 0]
    bp_r = bp.reshape(E, C, 1)
    gates_r = jnp.transpose(gates, (0, 2, 1))

    y0, y1, y2, y3, loss = pl.pallas_call(
        _moe_body,
        grid=(E,),
        in_specs=[
            pl.BlockSpec((B, C, HW), lambda e: (0, 0, 0)),
            pl.BlockSpec((G, E, C), lambda e: (0, 0, 0)),
            pl.BlockSpec((1, CCAP, 9 * C), lambda e: (e, 0, 0)),
            pl.BlockSpec((1, CCAP, 1), lambda e: (e, 0, 0)),
            pl.BlockSpec((1, C, CCAP), lambda e: (e, 0, 0)),
            pl.BlockSpec((1, C, 1), lambda e: (e, 0, 0)),
        ],
        out_specs=[
            pl.BlockSpec((B, C, HW), lambda e: (0, 0, 0)),
            pl.BlockSpec((B, C, HW), lambda e: (0, 0, 0)),
            pl.BlockSpec((B, C, HW), lambda e: (0, 0, 0)),
            pl.BlockSpec((B, C, HW), lambda e: (0, 0, 0)),
            pl.BlockSpec((1, 1), lambda e: (0, 0)),
        ],
        out_shape=[
            jax.ShapeDtypeStruct((B, C, HW), jnp.float32),
            jax.ShapeDtypeStruct((B, C, HW), jnp.float32),
            jax.ShapeDtypeStruct((B, C, HW), jnp.float32),
            jax.ShapeDtypeStruct((B, C, HW), jnp.float32),
            jax.ShapeDtypeStruct((1, 1), jnp.float32),
        ],
        scratch_shapes=[
            pltpu.VMEM((9 * C, BHW), jnp.bfloat16),
            pltpu.VMEM((E, G, BHW), jnp.float32),
        ],
        compiler_params=pltpu.CompilerParams(
            dimension_semantics=("arbitrary",),
        ),
    )(x_nat, gates_r, Wc_r, bc_r, Wp_r, bp_r)

    sh = (B, C, H, W)
    return (y0.reshape(sh), y1.reshape(sh), y2.reshape(sh), y3.reshape(sh),
            loss[0, 0])


def kernel(x, Wc, bc, Wp, bp, gates):
    return _moe(x, Wc, bc, Wp, bp, gates)
